# baseline (speedup 1.0000x reference)
import functools
import jax
import jax.numpy as jnp
from jax import lax
from jax.experimental import pallas as pl
from jax.experimental.pallas import tpu as pltpu
from jax.experimental.pallas import tpu_sc as plsc

_N = 10000
_E = 320000
_H = 64
_G = 128
_NOUT = 10

_NC, _NS = 2, 16
_NW = _NC * _NS
_CHUNK = 128
_EPW_CH = 80
_EPW = _CHUNK * _EPW_CH
_E_PAD = _NW * _EPW
_NPAD = 10240
_ROWS_PT = _NPAD // _NS


def _sc_agg_body(y_hbm, src_hbm, dst_hbm, zero_hbm, out_hbm,
                 src_v, dst_v, rows_v, sem_g, sem_s, table_sh):
    c = lax.axis_index("c")
    s = lax.axis_index("s")
    w = c * _NS + s

    pltpu.sync_copy(zero_hbm, table_sh.at[pl.ds(s * _ROWS_PT, _ROWS_PT)])
    pltpu.sync_copy(src_hbm.at[w], src_v)
    pltpu.sync_copy(dst_hbm.at[w], dst_v)
    plsc.subcore_barrier()

    pltpu.async_copy(y_hbm.at[src_v.at[0]], rows_v.at[0], sem_g)

    def step(j, _):
        b = lax.rem(j, 2)
        nb = 1 - b
        pltpu.make_async_copy(y_hbm.at[src_v.at[j]], rows_v.at[b], sem_g).wait()

        @pl.when(j < _EPW_CH - 1)
        def _():
            pltpu.async_copy(y_hbm.at[src_v.at[j + 1]], rows_v.at[nb], sem_g)

        pltpu.async_copy(rows_v.at[b], table_sh.at[dst_v.at[j]], sem_s,
                         add=True).wait()
        return 0

    lax.fori_loop(0, _EPW_CH, step, 0)
    plsc.subcore_barrier()
    pltpu.sync_copy(table_sh.at[pl.ds(s * _ROWS_PT, _ROWS_PT)],
                    out_hbm.at[c, pl.ds(s * _ROWS_PT, _ROWS_PT)])


@functools.lru_cache(maxsize=None)
def _build_sc_agg():
    return functools.partial(
        pl.kernel,
        out_type=jax.ShapeDtypeStruct((_NC, _NPAD, _H), jnp.float32),
        mesh=plsc.VectorSubcoreMesh(core_axis_name="c", subcore_axis_name="s",
                                    num_cores=_NC, num_subcores=_NS),
        compiler_params=pltpu.CompilerParams(use_tc_tiling_on_sc=False),
        scratch_types=[
            pltpu.VMEM((_EPW_CH, _CHUNK), jnp.int32),
            pltpu.VMEM((_EPW_CH, _CHUNK), jnp.int32),
            pltpu.VMEM((2, _CHUNK, _H), jnp.float32),
            pltpu.SemaphoreType.DMA,
            pltpu.SemaphoreType.DMA,
            pltpu.VMEM_SHARED((_NPAD, _H), jnp.float32),
        ],
    )(_sc_agg_body)


def _sc_agg(y, src_p, dst_p, zero_blk):
    return _build_sc_agg()(y, src_p, dst_p, zero_blk)


def _mm(a, b):
    return jnp.dot(a, b, preferred_element_type=jnp.float32,
                   precision=lax.Precision.HIGHEST)


def _tc0_body(x_ref, w_ref, o_ref):
    o_ref[:_N, :] = _mm(x_ref[...], w_ref[...])
    o_ref[_N:, :] = jnp.zeros((_NPAD - _N, _H), jnp.float32)


def _tc_mid_body(y_ref, p_ref, eps_ref, b1_ref, w2_ref, b2_ref,
                 g_ref, bb_ref, wn_ref, oy_ref, ohn_ref):
    y = y_ref[:_N, :]
    agg = p_ref[0, :_N, :] + p_ref[1, :_N, :]
    pre = (1.0 + eps_ref[0, 0]) * y + agg + b1_ref[...]
    h = _mm(jnp.maximum(pre, 0.0), w2_ref[...]) + b2_ref[...]
    h = jnp.maximum(h, 0.0)
    mu = jnp.mean(h, axis=0)
    var = jnp.mean(jnp.square(h - mu), axis=0)
    hn = (h - mu) * lax.rsqrt(var + 1e-5) * g_ref[...] + bb_ref[...]
    ohn_ref[...] = hn
    oy_ref[:_N, :] = _mm(hn, wn_ref[...])
    oy_ref[_N:, :] = jnp.zeros((_NPAD - _N, _H), jnp.float32)


def _tc_fin_body(hn_ref, batch_ref, fc1w_ref, fc1b_ref,
                 fc2w_ref, fc2b_ref, o_ref):
    hn = hn_ref[...]
    oh = (batch_ref[...] == lax.broadcasted_iota(jnp.int32, (_N, _G), 1)
          ).astype(jnp.float32)
    sums = lax.dot_general(oh, hn, (((0,), (0,)), ((), ())),
                           preferred_element_type=jnp.float32,
                           precision=lax.Precision.HIGHEST)
    counts = jnp.sum(oh, axis=0)
    pooled = sums / jnp.maximum(counts, 1.0)[:, None]
    t = _mm(pooled, fc1w_ref[...]) + fc1b_ref[...]
    t = jnp.where(t > 0.0, t, jnp.exp(t) - 1.0)
    logits = _mm(t, fc2w_ref[...]) + fc2b_ref[...]
    m = jnp.max(logits, axis=1, keepdims=True)
    lse = jnp.log(jnp.sum(jnp.exp(logits - m), axis=1, keepdims=True)) + m
    o_ref[...] = logits - lse


_tc0 = pl.pallas_call(
    _tc0_body, out_shape=jax.ShapeDtypeStruct((_NPAD, _H), jnp.float32))

_tc_mid = pl.pallas_call(
    _tc_mid_body,
    out_shape=(jax.ShapeDtypeStruct((_NPAD, _H), jnp.float32),
               jax.ShapeDtypeStruct((_N, _H), jnp.float32)))

_tc_fin = pl.pallas_call(
    _tc_fin_body, out_shape=jax.ShapeDtypeStruct((_G, _G), jnp.float32))


def kernel(x, edge_index, batch, eps1, eps2, eps3,
           nn1_W1, nn1_b1, nn1_W2, nn1_b2,
           nn2_W1, nn2_b1, nn2_W2, nn2_b2,
           nn3_W1, nn3_b1, nn3_W2, nn3_b2,
           bn1_g, bn1_b, bn2_g, bn2_b, bn3_g, bn3_b,
           fc1_W, fc1_b, fc2_W, fc2_b):
    pad = _E_PAD - _E
    src_p = jnp.concatenate(
        [edge_index[0], jnp.zeros((pad,), jnp.int32)]).reshape(_NW, _EPW_CH, _CHUNK)
    dst_p = jnp.concatenate(
        [edge_index[1], jnp.full((pad,), _N, jnp.int32)]).reshape(_NW, _EPW_CH, _CHUNK)
    zero_blk = jnp.zeros((_ROWS_PT, _H), jnp.float32)
    batch2 = batch.reshape(_N, 1)
    fc2_Wp = jnp.pad(fc2_W, ((0, 0), (0, _G - _NOUT)))
    fc2_bp = jnp.concatenate(
        [fc2_b, jnp.full((_G - _NOUT,), -1e30, jnp.float32)])

    epss = jnp.stack([eps1, eps2, eps3]).reshape(3, 1, 1)
    b1s = jnp.stack([nn1_b1, nn2_b1, nn3_b1])
    w2s = jnp.stack([nn1_W2, nn2_W2, nn3_W2])
    b2s = jnp.stack([nn1_b2, nn2_b2, nn3_b2])
    gs = jnp.stack([bn1_g, bn2_g, bn3_g])
    bbs = jnp.stack([bn1_b, bn2_b, bn3_b])
    wns = jnp.stack([nn2_W1, nn3_W1, nn2_W1])

    y1 = _tc0(x, nn1_W1)

    def body(y, params):
        eps_i, b1_i, w2_i, b2_i, g_i, bb_i, wn_i = params
        p = _sc_agg(y, src_p, dst_p, zero_blk)
        y_next, hn = _tc_mid(y, p, eps_i, b1_i, w2_i, b2_i, g_i, bb_i, wn_i)
        return y_next, hn

    _, hns = lax.scan(body, y1, (epss, b1s, w2s, b2s, gs, bbs, wns))
    out = _tc_fin(hns[-1], batch2, fc1_W, fc1_b, fc2_Wp, fc2_bp)
    return out[:, :_NOUT]


# speedup vs baseline: 2.4241x; 2.4241x over previous
import functools
import jax
import jax.numpy as jnp
from jax import lax
from jax.experimental import pallas as pl
from jax.experimental.pallas import tpu as pltpu
from jax.experimental.pallas import tpu_sc as plsc

_N = 10000
_E = 320000
_H = 64
_G = 128
_NOUT = 10

_NC, _NS = 2, 16
_NW = _NC * _NS
_CHUNK = 128
_EPW_CH = 80
_EPW = _CHUNK * _EPW_CH
_E_PAD = _NW * _EPW
_NPAD = 10240
_ROWS_PT = _NPAD // _NS


def _sc_agg_body(y_hbm, src_hbm, dst_hbm, zero_hbm, out_hbm,
                 src_v, dst_v, rows_v, sem_g, sem_s, y_sh, table_sh):
    c = lax.axis_index("c")
    s = lax.axis_index("s")
    w = c * _NS + s

    pltpu.sync_copy(y_hbm.at[pl.ds(s * _ROWS_PT, _ROWS_PT)],
                    y_sh.at[pl.ds(s * _ROWS_PT, _ROWS_PT)])
    pltpu.sync_copy(zero_hbm, table_sh.at[pl.ds(s * _ROWS_PT, _ROWS_PT)])
    pltpu.sync_copy(src_hbm.at[w], src_v)
    pltpu.sync_copy(dst_hbm.at[w], dst_v)
    plsc.subcore_barrier()

    pltpu.async_copy(y_sh.at[src_v.at[0]], rows_v.at[0], sem_g)

    def step(j, _):
        b = lax.rem(j, 2)
        nb = 1 - b
        pltpu.make_async_copy(y_sh.at[src_v.at[j]], rows_v.at[b], sem_g).wait()

        @pl.when(j < _EPW_CH - 1)
        def _():
            pltpu.async_copy(y_sh.at[src_v.at[j + 1]], rows_v.at[nb], sem_g)

        pltpu.async_copy(rows_v.at[b], table_sh.at[dst_v.at[j]], sem_s,
                         add=True).wait()
        return 0

    lax.fori_loop(0, _EPW_CH, step, 0)
    plsc.subcore_barrier()
    pltpu.sync_copy(table_sh.at[pl.ds(s * _ROWS_PT, _ROWS_PT)],
                    out_hbm.at[c, pl.ds(s * _ROWS_PT, _ROWS_PT)])


@functools.lru_cache(maxsize=None)
def _build_sc_agg():
    return functools.partial(
        pl.kernel,
        out_type=jax.ShapeDtypeStruct((_NC, _NPAD, _H), jnp.float32),
        mesh=plsc.VectorSubcoreMesh(core_axis_name="c", subcore_axis_name="s",
                                    num_cores=_NC, num_subcores=_NS),
        compiler_params=pltpu.CompilerParams(use_tc_tiling_on_sc=False),
        scratch_types=[
            pltpu.VMEM((_EPW_CH, _CHUNK), jnp.int32),
            pltpu.VMEM((_EPW_CH, _CHUNK), jnp.int32),
            pltpu.VMEM((2, _CHUNK, _H), jnp.float32),
            pltpu.SemaphoreType.DMA,
            pltpu.SemaphoreType.DMA,
            pltpu.VMEM_SHARED((_NPAD, _H), jnp.float32),
            pltpu.VMEM_SHARED((_NPAD, _H), jnp.float32),
        ],
    )(_sc_agg_body)


def _sc_agg(y, src_p, dst_p, zero_blk):
    return _build_sc_agg()(y, src_p, dst_p, zero_blk)


def _mm(a, b):
    return jnp.dot(a, b, preferred_element_type=jnp.float32,
                   precision=lax.Precision.HIGHEST)


_NP2 = _NPAD // 2
_N2 = _N // 2


def _tc0_body(x_ref, w_ref, o_ref):
    xv = x_ref[...]
    w = w_ref[...]
    ye = _mm(xv[:, :128], w)
    yo = _mm(xv[:, 128:], w)
    o_ref[:_N2, :] = jnp.concatenate([ye, yo], axis=1)
    o_ref[_N2:, :] = jnp.zeros((_NP2 - _N2, 2 * _H), jnp.float32)


def _tc_mid_body(y_ref, p_ref, eps_ref, b1_ref, w2_ref, b2_ref,
                 g_ref, bb_ref, wn_ref, oy_ref, ohn_ref):
    y = y_ref[:_N2, :]
    agg = p_ref[0, :_N2, :] + p_ref[1, :_N2, :]
    pre = (1.0 + eps_ref[0, 0]) * y + agg + b1_ref[...]
    h = _mm(jnp.maximum(pre, 0.0), w2_ref[...]) + b2_ref[...]
    h = jnp.maximum(h, 0.0)
    sh = jnp.sum(h, axis=0)
    mu64 = (sh[:_H] + sh[_H:]) / jnp.float32(_N)
    mu = jnp.concatenate([mu64, mu64])
    d = h - mu
    sv = jnp.sum(d * d, axis=0)
    var64 = (sv[:_H] + sv[_H:]) / jnp.float32(_N)
    var = jnp.concatenate([var64, var64])
    hn = d * lax.rsqrt(var + 1e-5) * g_ref[...] + bb_ref[...]
    ohn_ref[...] = hn
    oy_ref[:_N2, :] = _mm(hn, wn_ref[...])
    oy_ref[_N2:, :] = jnp.zeros((_NP2 - _N2, 2 * _H), jnp.float32)


def _tc_fin_body(hn_ref, batch_ref, fc1w_ref, fc1b_ref,
                 fc2w_ref, fc2b_ref, o_ref):
    hn = hn_ref[...]
    ohe = (batch_ref[:, 0:1] == lax.broadcasted_iota(jnp.int32, (_N2, _G), 1)
           ).astype(jnp.float32)
    oho = (batch_ref[:, 1:2] == lax.broadcasted_iota(jnp.int32, (_N2, _G), 1)
           ).astype(jnp.float32)
    dn = (((0,), (0,)), ((), ()))
    sums = (lax.dot_general(ohe, hn[:, :_H], dn,
                            preferred_element_type=jnp.float32,
                            precision=lax.Precision.HIGHEST) +
            lax.dot_general(oho, hn[:, _H:], dn,
                            preferred_element_type=jnp.float32,
                            precision=lax.Precision.HIGHEST))
    counts = jnp.sum(ohe, axis=0) + jnp.sum(oho, axis=0)
    pooled = sums / jnp.maximum(counts, 1.0)[:, None]
    t = _mm(pooled, fc1w_ref[...]) + fc1b_ref[...]
    t = jnp.where(t > 0.0, t, jnp.exp(t) - 1.0)
    logits = _mm(t, fc2w_ref[...]) + fc2b_ref[...]
    m = jnp.max(logits, axis=1, keepdims=True)
    lse = jnp.log(jnp.sum(jnp.exp(logits - m), axis=1, keepdims=True)) + m
    o_ref[...] = logits - lse


_tc0 = pl.pallas_call(
    _tc0_body, out_shape=jax.ShapeDtypeStruct((_NP2, 2 * _H), jnp.float32))

_tc_mid = pl.pallas_call(
    _tc_mid_body,
    out_shape=(jax.ShapeDtypeStruct((_NP2, 2 * _H), jnp.float32),
               jax.ShapeDtypeStruct((_N2, 2 * _H), jnp.float32)),
    input_output_aliases={0: 0})

_tc_fin = pl.pallas_call(
    _tc_fin_body, out_shape=jax.ShapeDtypeStruct((_G, _G), jnp.float32))


def kernel(x, edge_index, batch, eps1, eps2, eps3,
           nn1_W1, nn1_b1, nn1_W2, nn1_b2,
           nn2_W1, nn2_b1, nn2_W2, nn2_b2,
           nn3_W1, nn3_b1, nn3_W2, nn3_b2,
           bn1_g, bn1_b, bn2_g, bn2_b, bn3_g, bn3_b,
           fc1_W, fc1_b, fc2_W, fc2_b):
    pad = _E_PAD - _E
    src_p = jnp.concatenate(
        [edge_index[0], jnp.zeros((pad,), jnp.int32)]).reshape(_NW, _EPW_CH, _CHUNK)
    dst_p = jnp.concatenate(
        [edge_index[1], jnp.full((pad,), _N, jnp.int32)]).reshape(_NW, _EPW_CH, _CHUNK)
    zero_blk = jnp.zeros((_ROWS_PT, _H), jnp.float32)
    batch2 = batch.reshape(_N2, 2)
    fc2_Wp = jnp.pad(fc2_W, ((0, 0), (0, _G - _NOUT)))
    fc2_bp = jnp.concatenate(
        [fc2_b, jnp.full((_G - _NOUT,), -1e30, jnp.float32)])

    epss = jnp.stack([eps1, eps2, eps3]).reshape(3, 1, 1)

    def blkdiag(w):
        z = jnp.zeros((_H, _H), jnp.float32)
        return jnp.block([[w, z], [z, w]])

    def tile2(v):
        return jnp.concatenate([v, v])

    b1s = jnp.stack([tile2(nn1_b1), tile2(nn2_b1), tile2(nn3_b1)])
    w2s = jnp.stack([blkdiag(nn1_W2), blkdiag(nn2_W2), blkdiag(nn3_W2)])
    b2s = jnp.stack([tile2(nn1_b2), tile2(nn2_b2), tile2(nn3_b2)])
    gs = jnp.stack([tile2(bn1_g), tile2(bn2_g), tile2(bn3_g)])
    bbs = jnp.stack([tile2(bn1_b), tile2(bn2_b), tile2(bn3_b)])
    wns = jnp.stack([blkdiag(nn2_W1), blkdiag(nn3_W1), blkdiag(nn2_W1)])

    y1 = _tc0(x.reshape(_N2, 256), nn1_W1)

    def body(carry, params):
        y, _ = carry
        eps_i, b1_i, w2_i, b2_i, g_i, bb_i, wn_i = params
        p = _sc_agg(y.reshape(_NPAD, _H), src_p, dst_p, zero_blk)
        pp = p.reshape(_NC, _NP2, 2 * _H)
        y_next, hn = _tc_mid(y, pp, eps_i, b1_i, w2_i, b2_i, g_i, bb_i, wn_i)
        return (y_next, hn), None

    hn0 = jnp.zeros((_N2, 2 * _H), jnp.float32)
    (_, hn3), _ = lax.scan(body, (y1, hn0), (epss, b1s, w2s, b2s, gs, bbs, wns))
    out = _tc_fin(hn3, batch2, fc1_W, fc1_b, fc2_Wp, fc2_bp)
    return out[:, :_NOUT]
